# baseline (speedup 1.0000x reference)
import functools

import jax
import jax.numpy as jnp
from jax import lax
from jax.experimental import pallas as pl
from jax.experimental.pallas import tpu as pltpu
from jax.experimental.pallas import tpu_sc as plsc

_B, _T, _EMB = 4096, 200, 64
_CH0 = 104
_CH1 = _T - _CH0


def _sc_embed(x, table, pos):
    info = plsc.get_sparse_core_info()
    nw = info.num_cores * info.num_subcores
    rows_per_w = _B // nw

    mesh = plsc.VectorSubcoreMesh(core_axis_name="c", subcore_axis_name="s")

    @functools.partial(
        pl.kernel,
        out_type=jax.ShapeDtypeStruct((_B, _T, _EMB), jnp.float32),
        mesh=mesh,
        scratch_types=[
            pltpu.VMEM((_T,), jnp.int32),
            pltpu.VMEM((_T, _EMB), jnp.float32),
            pltpu.VMEM((_T, _EMB), jnp.float32),
            pltpu.SemaphoreType.DMA,
        ],
        compiler_params=pltpu.CompilerParams(use_tc_tiling_on_sc=False),
    )
    def k(x_hbm, table_hbm, pos_hbm, out_hbm, idx_v, rows_v, pos_v, sem):
        wid = lax.axis_index("s") * info.num_cores + lax.axis_index("c")
        base = wid * rows_per_w
        pltpu.sync_copy(pos_hbm, pos_v)

        @pl.loop(0, rows_per_w)
        def _rows(g):
            row = base + g
            pltpu.sync_copy(x_hbm.at[row], idx_v)
            pltpu.async_copy(
                table_hbm.at[idx_v.at[pl.ds(0, _CH0)]],
                rows_v.at[pl.ds(0, _CH0)],
                sem,
            ).wait()
            pltpu.async_copy(
                table_hbm.at[idx_v.at[pl.ds(_CH0, _CH1)]],
                rows_v.at[pl.ds(_CH0, _CH1)],
                sem,
            ).wait()

            @pl.loop(0, _T)
            def _add(t):
                for c in range(_EMB // 16):
                    sl = pl.ds(c * 16, 16)
                    rows_v[t, sl] = rows_v[t, sl] + pos_v[t, sl]

            pltpu.sync_copy(rows_v, out_hbm.at[row])

    return k(x, table, pos)


def kernel(x, input_table, pos_table, positions):
    pos = jnp.take(pos_table, positions, axis=0)
    return _sc_embed(x.astype(jnp.int32), input_table, pos)
